# Initial kernel scaffold; baseline (speedup 1.0000x reference)
import jax
import jax.numpy as jnp
from jax.experimental import pallas as pl


def kernel(x, mat1, mat2, mat3, mat4, sn1_W1, sn1_W2, sn1_b2, sn2_W1, sn2_W2, sn2_b2, sn3_W1, sn3_W2, sn3_b2, sn4_W1, sn4_W2, sn4_b2, conv5_W, lin1_W, lin2_W, lin3_W, lin3_b):
    raise NotImplementedError("write your pallas kernel here")


# speedup vs baseline: 2.9833x; 2.9833x over previous
import functools

import jax
import jax.numpy as jnp
import numpy as np
from jax import lax
from jax.experimental import pallas as pl
from jax.experimental.pallas import tpu as pltpu

B, N, K, M = 8, 1024, 20, 8
EPS = 1e-5
NEG = float("-inf")

CIN = (3, 64, 64, 128)
COUT = (64, 64, 128, 256)


def _dot(a, b):
    return lax.dot_general(a, b, (((1,), (0,)), ((), ())),
                           precision=lax.Precision.HIGHEST,
                           preferred_element_type=jnp.float32)


def _dotd(a, b):
    return lax.dot_general(a, b, (((1,), (0,)), ((), ())),
                           preferred_element_type=jnp.float32)


def _dotx(a, b):
    return lax.dot_general(a, b, (((1,), (0,)), ((), ())),
                           precision=lax.Precision.HIGHEST,
                           preferred_element_type=jnp.float32)


def _rowsum(u):
    return jnp.sum(u, axis=1, keepdims=True)


def _kernel_a(x_ref, w1b_ref, w2b_ref, b2t_ref, pt_ref, ekt_ref, gm_ref,
              idq_ref, sc1_ref, sc2_ref, sc3_ref, sc4_ref,
              xs_ref):
    sub = lax.broadcasted_iota(jnp.int32, (N, N), 0)
    eye = sub == lax.broadcasted_iota(jnp.int32, (N, N), 1)

    def knn_b(b, _):
        xb = x_ref[b]
        g2 = 2.0 * lax.dot_general(xb, xb, (((0,), (0,)), ((), ())),
                                   preferred_element_type=jnp.float32)
        xxr = jnp.sum(xb * xb, axis=0, keepdims=True)
        xxc = _rowsum(jnp.where(eye, jnp.zeros((N, N), jnp.float32) + xxr,
                                0.0))

        def step(k, carry):
            wpd, xs_acc, idf = carry
            mx = jnp.max(wpd, axis=0, keepdims=True)
            selT = jnp.min(jnp.where(wpd == mx, sub, N), axis=0,
                           keepdims=True)
            ohb = sub == selT
            ohf = ohb.astype(jnp.float32)
            gT = _dot(xb, ohf)
            cat6 = jnp.concatenate([gT - xb, gT], axis=0)
            xs_acc = xs_acc + _dot(pt_ref[k], cat6)
            idf = idf + _dotx(ekt_ref[k], selT.astype(jnp.float32))
            return jnp.where(ohb, NEG, wpd), xs_acc, idf

        _, xs_acc, idf = lax.fori_loop(
            0, K, step,
            ((g2 - xxc) - xxr, jnp.zeros((120, N), jnp.float32),
             jnp.zeros((32, N), jnp.float32)),
            unroll=False)
        xs_ref[b] = xs_acc
        idq_ref[b] = idf
        return 0

    lax.fori_loop(0, B, knn_b, 0, unroll=False)

    means = []
    rstds = []
    for i in range(4):
        def acc_b(b, c, i=i):
            u = _dotd(w1b_ref[i], xs_ref[b])
            return c[0] + _rowsum(u), c[1] + _rowsum(u * u)

        s1, s2 = lax.fori_loop(
            0, B, acc_b,
            (jnp.zeros((320, 1), jnp.float32), jnp.zeros((320, 1), jnp.float32)),
            unroll=False)
        cs = sum([s1[16 * k:16 * k + 16] for k in range(K)],
                 jnp.zeros((16, 1), jnp.float32))
        cq = sum([s2[16 * k:16 * k + 16] for k in range(K)],
                 jnp.zeros((16, 1), jnp.float32))
        cnt = jnp.float32(B * N * K)
        mean = cs / cnt
        var = cq / cnt - mean * mean
        means.append(jnp.concatenate([mean] * K, axis=0))
        rstds.append(jnp.concatenate([lax.rsqrt(var + EPS)] * K, axis=0))

    sc_refs = (sc1_ref, sc2_ref, sc3_ref, sc4_ref)

    def pass2_b(b, _):
        xs = xs_ref[b]
        for i in range(4):
            u = _dotd(w1b_ref[i], xs)
            h = jax.nn.relu((u - means[i]) * rstds[i])
            s = _dotd(w2b_ref[i], h) + b2t_ref[i]
            e = jnp.exp(s - jnp.max(s, axis=0, keepdims=True))
            den = _dot(gm_ref[...], e)
            sc_refs[i][b] = e / den + 0.5
        return 0

    lax.fori_loop(0, B, pass2_b, 0, unroll=False)


def _call_kernel_a(x, w1b, w2b, b2t, pt, ekt, gm):
    return pl.pallas_call(
        _kernel_a,
        out_shape=(
            jax.ShapeDtypeStruct((B, 32, N), jnp.float32),
            jax.ShapeDtypeStruct((B, 160, N), jnp.float32),
            jax.ShapeDtypeStruct((B, 160, N), jnp.float32),
            jax.ShapeDtypeStruct((B, 160, N), jnp.float32),
            jax.ShapeDtypeStruct((B, 160, N), jnp.float32),
        ),
        scratch_shapes=[pltpu.VMEM((B, 120, N), jnp.float32)],
        compiler_params=pltpu.CompilerParams(vmem_limit_bytes=63 * 2**20),
    )(x, w1b, w2b, b2t, pt, ekt, gm)


def _kernel_b(idq_ref, sc_ref, pp_ref, k2v_ref, k1v_ref, ek1_ref, e8_ref,
              r_ref, s_ref, p_ref, o_ref, *, c, o):
    subf = lax.broadcasted_iota(jnp.int32, (N, N), 0).astype(jnp.float32)

    def body_b(b, carry):
        xp = pp_ref[b]
        idq = idq_ref[b]
        scb = sc_ref[b]

        def step(k, kc):
            am_all, ss8 = kc
            selT = _dotx(ek1_ref[k], idq)
            ohf = (subf == selT).astype(jnp.float32)
            gk = _dot(xp, ohf)
            cols8 = _dot(e8_ref[k], scb)
            am_all = am_all + _dot(r_ref[...], cols8) * _dot(s_ref[...], gk)
            return am_all, ss8 + cols8

        am_all, ss8 = lax.fori_loop(
            0, K, step,
            (jnp.zeros((M * c, N), jnp.float32),
             jnp.zeros((M, N), jnp.float32)),
            unroll=False)
        q_all = _dot(r_ref[...], ss8) * _dot(s_ref[...], xp)
        ob = _dot(k2v_ref[...], am_all) - _dot(k1v_ref[...], q_all)
        o_ref[b] = ob
        return carry[0] + _rowsum(ob), carry[1] + _rowsum(ob * ob)

    osum, osq = lax.fori_loop(
        0, B, body_b,
        (jnp.zeros((o, 1), jnp.float32), jnp.zeros((o, 1), jnp.float32)),
        unroll=False)
    cnt = jnp.float32(B * N)
    mean = osum / cnt
    rstd = lax.rsqrt(osq / cnt - mean * mean + EPS)

    def norm_b(b, _):
        p_ref[b] = jax.nn.relu((o_ref[b] - mean) * rstd)
        return 0

    lax.fori_loop(0, B, norm_b, 0, unroll=False)


def _call_kernel_b(idq, sc, pp, k2v, k1v, ek1, e8, r, s, c, o):
    return pl.pallas_call(
        functools.partial(_kernel_b, c=c, o=o),
        out_shape=jax.ShapeDtypeStruct((B, o, N), jnp.float32),
        scratch_shapes=[pltpu.VMEM((B, o, N), jnp.float32)],
        compiler_params=pltpu.CompilerParams(vmem_limit_bytes=63 * 2**20),
    )(idq, sc, pp, k2v, k1v, ek1, e8, r, s)


def _kernel_d(p1_ref, p2_ref, p3_ref, p4_ref, c5_ref, l1_ref, l2_ref,
              l3_ref, l3b_ref, out_ref):
    def conv_b(b):
        feat = jnp.concatenate(
            [p1_ref[b], p2_ref[b], p3_ref[b], p4_ref[b]], axis=0)
        return _dot(c5_ref[...], feat)

    def stat_b(b, carry):
        cb = conv_b(b)
        return carry[0] + _rowsum(cb), carry[1] + _rowsum(cb * cb)

    osum, osq = lax.fori_loop(
        0, B, stat_b,
        (jnp.zeros((1024, 1), jnp.float32), jnp.zeros((1024, 1), jnp.float32)),
        unroll=False)
    cnt = jnp.float32(B * N)
    mean = osum / cnt
    rstd = lax.rsqrt(osq / cnt - mean * mean + EPS)

    lane8 = lax.broadcasted_iota(jnp.int32, (1, B), 1)

    def pool_b(b, acc):
        z = jax.nn.relu((conv_b(b) - mean) * rstd)
        col = jnp.concatenate(
            [jnp.max(z, axis=1, keepdims=True), _rowsum(z) * (1.0 / N)],
            axis=0)
        return acc + _dot(col, (lane8 == b).astype(jnp.float32))

    fv = lax.fori_loop(0, B, pool_b, jnp.zeros((2048, B), jnp.float32),
                       unroll=False)

    def bn0_relu(h):
        mu = _rowsum(h) / B
        vr = _rowsum(h * h) / B - mu * mu
        return jax.nn.relu((h - mu) * lax.rsqrt(vr + EPS))

    h1 = bn0_relu(_dot(l1_ref[...], fv))
    h2 = bn0_relu(_dot(l2_ref[...], h1))
    out_ref[...] = _dot(l3_ref[...], h2) + l3b_ref[...]


def _call_kernel_d(p1, p2, p3, p4, c5w, l1, l2, l3, l3b):
    return pl.pallas_call(
        _kernel_d,
        out_shape=jax.ShapeDtypeStruct((40, B), jnp.float32),
        compiler_params=pltpu.CompilerParams(vmem_limit_bytes=63 * 2**20),
    )(p1, p2, p3, p4, c5w, l1, l2, l3, l3b)


def _stack_kernels(mat, c, o):
    r = mat.reshape(2 * c, M, o)
    k2 = (r[:c] + r[c:]).transpose(1, 0, 2).reshape(M * c, o)
    k1 = r[:c].transpose(1, 0, 2).reshape(M * c, o)
    return k2.T, k1.T


def _blockdiag(w, nblk):
    a, b = w.shape
    eye = jnp.eye(nblk, dtype=w.dtype)
    return (eye[:, None, :, None] * w[None, :, None, :]).reshape(nblk * a, nblk * b)


def kernel(x, mat1, mat2, mat3, mat4, sn1_W1, sn1_W2, sn1_b2, sn2_W1, sn2_W2, sn2_b2, sn3_W1, sn3_W2, sn3_b2, sn4_W1, sn4_W2, sn4_b2, conv5_W, lin1_W, lin2_W, lin3_W, lin3_b):
    mats = (mat1, mat2, mat3, mat4)
    sns = ((sn1_W1, sn1_W2, sn1_b2), (sn2_W1, sn2_W2, sn2_b2),
           (sn3_W1, sn3_W2, sn3_b2), (sn4_W1, sn4_W2, sn4_b2))

    w1b = jnp.stack([_blockdiag(w1, K) for (w1, _, _) in sns])
    w2b = jnp.stack([_blockdiag(w2, K) for (_, w2, _) in sns])
    b2t = jnp.stack([jnp.tile(b2, K)[:, None] for (_, _, b2) in sns])

    k2v = []
    k1v = []
    for i in range(4):
        a, b = _stack_kernels(mats[i], CIN[i], COUT[i])
        k2v.append(a)
        k1v.append(b)

    pt = np.zeros((K, 120, 6), np.float32)
    ekt = np.zeros((K, 32, 1), np.float32)
    ek1 = np.zeros((K, 1, 32), np.float32)
    e8 = np.zeros((K, 8, 160), np.float32)
    for k in range(K):
        for ci in range(6):
            pt[k, 6 * k + ci, ci] = 1.0
        ekt[k, k, 0] = 1.0
        ek1[k, 0, k] = 1.0
        for m in range(M):
            e8[k, m, 8 * k + m] = 1.0
    gm = np.kron(np.eye(K, dtype=np.float32), np.ones((M, M), np.float32))
    rs = []
    ss = []
    for c in CIN:
        r = np.zeros((M * c, M), np.float32)
        s = np.zeros((M * c, c), np.float32)
        for m in range(M):
            for j in range(c):
                r[m * c + j, m] = 1.0
                s[m * c + j, j] = 1.0
        rs.append(jnp.asarray(r))
        ss.append(jnp.asarray(s))
    pt, ekt, ek1, e8, gm = (jnp.asarray(v) for v in (pt, ekt, ek1, e8, gm))

    idq, sc1, sc2, sc3, sc4 = _call_kernel_a(x, w1b, w2b, b2t, pt, ekt, gm)

    scs = (sc1, sc2, sc3, sc4)
    p = x
    ps = []
    for i in range(4):
        p = _call_kernel_b(idq, scs[i], p, k2v[i], k1v[i], ek1, e8,
                           rs[i], ss[i], CIN[i], COUT[i])
        ps.append(p)

    out = _call_kernel_d(ps[0], ps[1], ps[2], ps[3], conv5_W, lin1_W,
                         lin2_W, lin3_W, lin3_b[:, None])
    return out.T
